# baseline (speedup 1.0000x reference)
import functools

import jax
import jax.numpy as jnp
from jax import lax
from jax.experimental import pallas as pl
from jax.experimental.pallas import tpu as pltpu
from jax.experimental.pallas import tpu_sc as plsc

F = 64
S = 200
B = 4096
NC = 2
NS = 16
NW = NC * NS
TOTAL = B * S
SEQ_PER_W = TOTAL // NW // S
LANES = 16


def _emb_body(ids_hbm, table_hbm, pos_hbm, out_hbm, idx_v, rows_v, pos_v, gsem):
    wid = lax.axis_index("s") * NC + lax.axis_index("c")
    pltpu.sync_copy(pos_hbm, pos_v)
    base = wid * SEQ_PER_W

    def do_chunk(c, carry):
        row0 = (base + c) * S
        pltpu.sync_copy(ids_hbm.at[pl.ds(row0, S)], idx_v)
        pltpu.async_copy(table_hbm.at[idx_v], rows_v, gsem).wait()

        def add_row(r, carry2):
            for j in range(F // LANES):
                sl = pl.ds(j * LANES, LANES)
                plsc.addupdate(rows_v.at[r, sl], pos_v[r, sl])
            return carry2

        lax.fori_loop(0, S, add_row, 0)
        pltpu.sync_copy(rows_v, out_hbm.at[pl.ds(row0, S)])
        return carry

    lax.fori_loop(0, SEQ_PER_W, do_chunk, 0)


_emb = functools.partial(
    pl.kernel,
    out_type=jax.ShapeDtypeStruct((TOTAL, F), jnp.float32),
    mesh=plsc.VectorSubcoreMesh(core_axis_name="c", subcore_axis_name="s"),
    scratch_types=[
        pltpu.VMEM((S,), jnp.int32),
        pltpu.VMEM((S, F), jnp.float32),
        pltpu.VMEM((S, F), jnp.float32),
        pltpu.SemaphoreType.DMA,
    ],
    compiler_params=pltpu.CompilerParams(use_tc_tiling_on_sc=False),
)(_emb_body)


def kernel(input_ids, input_embedding_weight, position_embedding):
    ids = input_ids.reshape(-1).astype(jnp.int32)
    out = _emb(ids, input_embedding_weight, position_embedding)
    return out.reshape(B, S, F)


# speedup vs baseline: 1.1929x; 1.1929x over previous
import functools

import jax
import jax.numpy as jnp
from jax import lax
from jax.experimental import pallas as pl
from jax.experimental.pallas import tpu as pltpu
from jax.experimental.pallas import tpu_sc as plsc

F = 64
S = 200
B = 4096
NC = 2
NS = 16
NW = NC * NS
TOTAL = B * S
PER_W = TOTAL // NW
N_CHUNKS = PER_W // S
NBUF = 4
LANES = 16


def _emb_body(ids_hbm, table_hbm, pos_hbm, out_hbm, idx_all, pos_v,
              r0, r1, r2, r3, g0, g1, g2, g3, o0, o1, o2, o3):
    rows = (r0, r1, r2, r3)
    gsem = (g0, g1, g2, g3)
    osem = (o0, o1, o2, o3)
    wid = lax.axis_index("s") * NC + lax.axis_index("c")
    base_row = wid * PER_W
    pltpu.sync_copy(pos_hbm, pos_v)
    pltpu.sync_copy(ids_hbm.at[pl.ds(base_row, PER_W)], idx_all)

    def gather_start(c, b):
        pltpu.async_copy(table_hbm.at[idx_all.at[pl.ds(c * S, S)]],
                         rows[b], gsem[b])

    def gather_wait(c, b):
        pltpu.make_async_copy(table_hbm.at[idx_all.at[pl.ds(c * S, S)]],
                              rows[b], gsem[b]).wait()

    def write_start(c, b):
        pltpu.async_copy(rows[b], out_hbm.at[pl.ds(base_row + c * S, S)],
                         osem[b])

    def write_wait(c, b):
        pltpu.make_async_copy(rows[b], out_hbm.at[pl.ds(base_row + c * S, S)],
                              osem[b]).wait()

    for b in range(NBUF):
        gather_start(b, b)

    def outer(i, carry):
        for b in range(NBUF):
            c = i * NBUF + b
            gather_wait(c, b)

            @plsc.parallel_loop(0, S, unroll=2)
            def _add(r):
                for j in range(F // LANES):
                    sl = pl.ds(j * LANES, LANES)
                    plsc.addupdate(rows[b].at[r, sl], pos_v[r, sl])

            write_start(c, b)

            bb = (b + 2) % NBUF

            @pl.when(jnp.logical_and(c >= 2, c + 2 < N_CHUNKS))
            def _next():
                write_wait(c - 2, bb)
                gather_start(c + 2, bb)

        return carry

    lax.fori_loop(0, N_CHUNKS // NBUF, outer, 0)
    for b in range(NBUF):
        write_wait(N_CHUNKS - NBUF + b, b)


_emb = functools.partial(
    pl.kernel,
    out_type=jax.ShapeDtypeStruct((TOTAL, F), jnp.float32),
    mesh=plsc.VectorSubcoreMesh(core_axis_name="c", subcore_axis_name="s"),
    scratch_types=[
        pltpu.VMEM((PER_W,), jnp.int32),
        pltpu.VMEM((S, F), jnp.float32),
    ] + [pltpu.VMEM((S, F), jnp.float32) for _ in range(NBUF)]
      + [pltpu.SemaphoreType.DMA for _ in range(2 * NBUF)],
    compiler_params=pltpu.CompilerParams(use_tc_tiling_on_sc=False),
)(_emb_body)


def kernel(input_ids, input_embedding_weight, position_embedding):
    ids = input_ids.reshape(-1).astype(jnp.int32)
    out = _emb(ids, input_embedding_weight, position_embedding)
    return out.reshape(B, S, F)
